# baseline (speedup 1.0000x reference)
import jax
import jax.numpy as jnp
from jax.experimental import pallas as pl
from jax.experimental.pallas import tpu as pltpu

B, N, K, C = 4, 1024, 20, 50
BN_EPS = 1e-5
_INTERPRET = False


def _mmT(a, w):
    return jax.lax.dot_general(
        a.astype(jnp.bfloat16), w.astype(jnp.bfloat16),
        (((1,), (1,)), ((), ())), preferred_element_type=jnp.float32)


def _mm(a, b):
    return jax.lax.dot_general(
        a.astype(jnp.bfloat16), b.astype(jnp.bfloat16),
        (((1,), (0,)), ((), ())), preferred_element_type=jnp.float32)


def _bn_s(g):
    return g / jnp.sqrt(jnp.asarray(1.0 + BN_EPS, jnp.float32))


def _cbr(h, W, cb, g, bb, relu=True):
    y = (_mmT(h, W) + cb) * _bn_s(g) + bb
    return jnp.maximum(y, 0.0) if relu else y


def _argmax_lanes(v):
    n, c = v.shape
    m = jnp.max(v, axis=1, keepdims=True)
    iota = jax.lax.broadcasted_iota(jnp.int32, (n, c), 1)
    cand = jnp.where(v == m, iota, c)
    return jnp.min(cand, axis=1, keepdims=True)


def _tnet_conv(h, ref, pref):
    a = _cbr(h, ref[pref + 'c1W'][...], ref[pref + 'c1b'][...],
             ref[pref + 'bn1g'][...], ref[pref + 'bn1b'][...])
    a = _cbr(a, ref[pref + 'c2W'][...], ref[pref + 'c2b'][...],
             ref[pref + 'bn2g'][...], ref[pref + 'bn2b'][...])
    a = _cbr(a, ref[pref + 'c3W'][...], ref[pref + 'c3b'][...],
             ref[pref + 'bn3g'][...], ref[pref + 'bn3b'][...])
    return jnp.max(a, axis=0)


def _p1_kernel(refs_by_name, x_ref, tmax_ref):
    h = x_ref[0]
    tmax_ref[0, 0, :] = _tnet_conv(h, refs_by_name, 't3_')


def _fc_head(t, ref, pref, eye_flat):
    h = jnp.maximum((_mmT(t, ref[pref + 'f1W'][...]) + ref[pref + 'f1b'][...])
                    * _bn_s(ref[pref + 'fbn1g'][...]) + ref[pref + 'fbn1b'][...], 0.0)
    h = jnp.maximum((_mmT(h, ref[pref + 'f2W'][...]) + ref[pref + 'f2b'][...])
                    * _bn_s(ref[pref + 'fbn2g'][...]) + ref[pref + 'fbn2b'][...], 0.0)
    h = _mmT(h, ref[pref + 'f3W'][...]) + ref[pref + 'f3b'][...]
    return h + eye_flat


def _p2_kernel(refs_by_name, tmax_ref, out_ref):
    iota = jax.lax.broadcasted_iota(jnp.int32, (B, 9), 1)
    eye_flat = jnp.where((iota == 0) | (iota == 4) | (iota == 8), 1.0, 0.0)
    out_ref[...] = _fc_head(tmax_ref[:, 0, :], refs_by_name, 't3_', eye_flat)


def _p4_kernel(refs_by_name, tmax_ref, out_ref):
    k2 = out_ref.shape[1]
    iota = jax.lax.broadcasted_iota(jnp.int32, (B, k2), 1)
    eye_flat = jnp.where((iota >> 7) == (iota & 127), 1.0, 0.0)
    out_ref[...] = _fc_head(tmax_ref[:, 0, :], refs_by_name, 'tk_', eye_flat)


def _p3_kernel(refs_by_name, x_ref, tr_ref, out1_ref, out2_ref, out3_ref, tmax2_ref):
    h = _mm(x_ref[0], tr_ref[0])
    o1 = _cbr(h, refs_by_name['c1W'][...], refs_by_name['c1b'][...],
              refs_by_name['c1g'][...], refs_by_name['c1bb'][...])
    o2 = _cbr(o1, refs_by_name['c2W'][...], refs_by_name['c2b'][...],
              refs_by_name['c2g'][...], refs_by_name['c2bb'][...])
    o3 = _cbr(o2, refs_by_name['c3W'][...], refs_by_name['c3b'][...],
              refs_by_name['c3g'][...], refs_by_name['c3bb'][...])
    out1_ref[0] = o1
    out2_ref[0] = o2
    out3_ref[0] = o3
    tmax2_ref[0, 0, :] = _tnet_conv(o3, refs_by_name, 'tk_')


def _p5_kernel(refs_by_name, out3_ref, trkd_ref, out4_ref, out5_ref, lf_ref):
    h = _mm(out3_ref[0], trkd_ref[0])
    o4 = _cbr(h, refs_by_name['c4W'][...], refs_by_name['c4b'][...],
              refs_by_name['c4g'][...], refs_by_name['c4bb'][...])
    o5 = _cbr(o4, refs_by_name['c5W'][...], refs_by_name['c5b'][...],
              refs_by_name['c5g'][...], refs_by_name['c5bb'][...], relu=False)
    hc = _cbr(o5, refs_by_name['k1W'][...], refs_by_name['k1b'][...],
              refs_by_name['k1g'][...], refs_by_name['k1bb'][...])
    hc = _cbr(hc, refs_by_name['k2W'][...], refs_by_name['k2b'][...],
              refs_by_name['k2g'][...], refs_by_name['k2bb'][...])
    lf = _mmT(hc, refs_by_name['k3W'][...]) + refs_by_name['k3b'][...]
    out4_ref[0] = o4
    out5_ref[0] = o5
    lf_ref[0] = lf


_NBLK = 8
_BW = N // _NBLK
_CW = _NBLK * K
_CPAD = 256


def _knn_accum(f, dd_ref, dorig_ref, cv_ref, ci_ref, mask_ref):
    sq = jnp.sum(f * f, axis=1, keepdims=True)
    ff = _mmT(f, f)
    d = sq + jnp.transpose(sq) - 2.0 * ff
    iota_r = jax.lax.broadcasted_iota(jnp.int32, (N, N), 0)
    iota_c = jax.lax.broadcasted_iota(jnp.int32, (N, N), 1)
    dorig_ref[...] = jnp.where(iota_r == iota_c, jnp.inf, d)
    dd_ref[...] = dorig_ref[...]
    cv_ref[...] = jnp.full((N, _CPAD), jnp.inf, jnp.float32)
    ci_ref[...] = jnp.full((N, _CPAD), N, jnp.int32)

    iota_b = jax.lax.broadcasted_iota(jnp.int32, (N, _BW), 1)
    iota_cand = jax.lax.broadcasted_iota(jnp.int32, (N, _CPAD), 1)

    for bb in range(_NBLK):
        lo = bb * _BW

        def blk_body(it, carry, lo=lo, bb=bb):
            blk = dd_ref[:, lo:lo + _BW]
            m = jnp.min(blk, axis=1, keepdims=True)
            j = jnp.min(jnp.where(blk == m, iota_b, _BW), axis=1, keepdims=True)
            dd_ref[:, lo:lo + _BW] = jnp.where(iota_b == j, jnp.inf, blk)
            col = bb * K + it
            cv_ref[...] = jnp.where(iota_cand == col, m, cv_ref[...])
            ci_ref[...] = jnp.where(iota_cand == col, j + lo, ci_ref[...])
            return carry

        jax.lax.fori_loop(0, K, blk_body, 0)

    def merge_body(_, carry):
        vals, _last = carry
        m = jnp.min(vals, axis=1, keepdims=True)
        pos = jnp.min(jnp.where(vals == m, iota_cand, _CPAD), axis=1, keepdims=True)
        return jnp.where(iota_cand == pos, jnp.inf, vals), m

    _, vK = jax.lax.fori_loop(
        0, K, merge_body, (cv_ref[...], jnp.zeros((N, 1), jnp.float32)))

    dorig = dorig_ref[...]
    c_lt = jnp.sum(jnp.where(dorig < vK, 1, 0), axis=1, keepdims=True)
    t_need = K - c_lt

    def jlast_body(t, carry):
        jl, idxe = carry
        cur = jnp.min(idxe, axis=1, keepdims=True)
        jl = jnp.where(t < t_need, cur, jl)
        return jl, jnp.where(idxe == cur, N, idxe)

    idxe0 = jnp.where(cv_ref[...] == vK, ci_ref[...], N)
    jlast, _ = jax.lax.fori_loop(
        0, K, jlast_body, (jnp.full((N, 1), -1, jnp.int32), idxe0))

    selm = jnp.logical_or(
        dorig < vK,
        jnp.logical_and(jnp.logical_and(dorig == vK, iota_c <= jlast),
                        jnp.isinf(dd_ref[...])))
    mask_ref[...] = jnp.where(selm, 1.0, mask_ref[...])


def _p6_kernel(x_ref, out5_ref, lf_ref, nrm_ref, gcnW_ref, gcnb_ref,
               lf2_ref, probs_ref, dd_ref, dorig_ref, mask_ref, w_ref,
               cv_ref, ci_ref):
    mask_ref[...] = jnp.zeros((N, N), jnp.float32)
    _knn_accum(x_ref[0], dd_ref, dorig_ref, cv_ref, ci_ref, mask_ref)
    _knn_accum(out5_ref[0], dd_ref, dorig_ref, cv_ref, ci_ref, mask_ref)
    mq = mask_ref[...]

    lf = lf_ref[0]
    label = _argmax_lanes(lf)
    iota_cls = jax.lax.broadcasted_iota(jnp.int32, (N, C), 1)
    oh = jnp.where(iota_cls == label, 1.0, 0.0)
    votes = _mm(mq, oh)
    freq = _argmax_lanes(votes)

    vm = jnp.logical_and(mq > 0.0, jnp.transpose(label) == freq)

    nf = nrm_ref[0]
    n0, n1, n2 = nf[:, 0:1], nf[:, 1:2], nf[:, 2:3]
    num = (n0 * jnp.transpose(n0) + n1 * jnp.transpose(n1)
           + n2 * jnp.transpose(n2))
    nrm = jnp.maximum(jnp.sqrt(n0 * n0 + n1 * n1 + n2 * n2), 1e-8)
    ew = jnp.abs(num / (nrm * jnp.transpose(nrm)))

    iota_r = jax.lax.broadcasted_iota(jnp.int32, (N, N), 0)
    iota_c = jax.lax.broadcasted_iota(jnp.int32, (N, N), 1)
    wd = jnp.where(vm, ew, 0.0) + jnp.where(iota_r == iota_c, 1.0, 0.0)
    w_ref[...] = wd
    deg = jnp.sum(wd, axis=1, keepdims=True)
    dinv = jnp.where(deg > 0.0, 1.0 / jnp.sqrt(deg), 0.0)
    wn = (w_ref[...] * jnp.transpose(dinv)) * dinv

    xw = _mmT(lf, gcnW_ref[...])
    out = jax.lax.dot_general(wn, xw, (((1,), (0,)), ((), ())),
                              preferred_element_type=jnp.float32,
                              precision=jax.lax.Precision.HIGHEST)
    lf2 = out + gcnb_ref[...]
    lf2_ref[0] = lf2
    fin = _argmax_lanes(lf2)
    probs_ref[0] = jnp.where(iota_cls == fin, 1.0, 0.0)


_P1_NAMES = tuple('t3_' + s for s in (
    'c1W', 'c1b', 'bn1g', 'bn1b', 'c2W', 'c2b', 'bn2g', 'bn2b',
    'c3W', 'c3b', 'bn3g', 'bn3b'))
_P2_NAMES = tuple('t3_' + s for s in (
    'f1W', 'f1b', 'fbn1g', 'fbn1b', 'f2W', 'f2b', 'fbn2g', 'fbn2b', 'f3W', 'f3b'))
_P3_NAMES = ('c1W', 'c1b', 'c1g', 'c1bb', 'c2W', 'c2b', 'c2g', 'c2bb',
             'c3W', 'c3b', 'c3g', 'c3bb') + tuple('tk_' + s for s in (
    'c1W', 'c1b', 'bn1g', 'bn1b', 'c2W', 'c2b', 'bn2g', 'bn2b',
    'c3W', 'c3b', 'bn3g', 'bn3b'))
_P4_NAMES = tuple('tk_' + s for s in (
    'f1W', 'f1b', 'fbn1g', 'fbn1b', 'f2W', 'f2b', 'fbn2g', 'fbn2b', 'f3W', 'f3b'))
_P5_NAMES = ('c4W', 'c4b', 'c4g', 'c4bb', 'c5W', 'c5b', 'c5g', 'c5bb',
             'k1W', 'k1b', 'k1g', 'k1bb', 'k2W', 'k2b', 'k2g', 'k2bb',
             'k3W', 'k3b')


def _prep(p, names):
    out = {}
    for nm in names:
        v = p[nm]
        out[nm] = v.reshape(1, -1) if v.ndim == 1 else v
    return out


def _named_kernel(body, names):
    def k(*refs):
        by_name = dict(zip(names, refs[:len(names)]))
        body(by_name, *refs[len(names):])
    return k


def _full_spec(v):
    nd = v.ndim
    return pl.BlockSpec(v.shape, lambda *_: (0,) * nd)


def _batch_spec(shape):
    return pl.BlockSpec((1,) + shape[1:], lambda b: (b,) + (0,) * (len(shape) - 1))


def _call_grid(body, names, wdict, ins, outs):
    wvals = [wdict[nm] for nm in names]
    return pl.pallas_call(
        _named_kernel(body, names),
        grid=(B,),
        in_specs=[_full_spec(v) for v in wvals] + [_batch_spec(v.shape) for v in ins],
        out_specs=[_batch_spec(s.shape) for s in outs],
        out_shape=outs,
        interpret=_INTERPRET,
    )(*wvals, *ins)


def _call_flat(body, names, wdict, ins, outs):
    wvals = [wdict[nm] for nm in names]
    return pl.pallas_call(
        _named_kernel(body, names),
        in_specs=[_full_spec(v) for v in wvals] + [_full_spec(v) for v in ins],
        out_specs=[_full_spec(s) for s in outs],
        out_shape=outs,
        interpret=_INTERPRET,
    )(*wvals, *ins)


def kernel(x, normals, params):
    p = params
    f32 = jnp.float32
    sds = jax.ShapeDtypeStruct

    (tmax,) = _call_grid(_p1_kernel, _P1_NAMES, _prep(p, _P1_NAMES), [x],
                         [sds((B, 1, 1024), f32)])
    (tr9,) = _call_flat(_p2_kernel, _P2_NAMES, _prep(p, _P2_NAMES), [tmax],
                        [sds((B, 9), f32)])
    trans3d = tr9.reshape(B, 3, 3)
    out1, out2, out3, tmax2 = _call_grid(
        _p3_kernel, _P3_NAMES, _prep(p, _P3_NAMES), [x, trans3d],
        [sds((B, N, 64), f32), sds((B, N, 128), f32), sds((B, N, 128), f32),
         sds((B, 1, 1024), f32)])
    (trkd,) = _call_flat(_p4_kernel, _P4_NAMES, _prep(p, _P4_NAMES), [tmax2],
                         [sds((B, 16384), f32)])
    transkd = trkd.reshape(B, 128, 128)
    out4, out5, lf = _call_grid(
        _p5_kernel, _P5_NAMES, _prep(p, _P5_NAMES), [out3, transkd],
        [sds((B, N, 256), f32), sds((B, N, 512), f32), sds((B, N, C), f32)])
    gcnW = p['gcnW']
    gcnb = p['gcnb'].reshape(1, C)
    ins = [x, out5, lf, normals, gcnW, gcnb]
    lf2, probs = pl.pallas_call(
        _p6_kernel,
        grid=(B,),
        in_specs=[_batch_spec(x.shape), _batch_spec(out5.shape),
                  _batch_spec(lf.shape), _batch_spec(normals.shape),
                  _full_spec(gcnW), _full_spec(gcnb)],
        out_specs=[_batch_spec((B, N, C)), _batch_spec((B, N, C))],
        out_shape=[sds((B, N, C), f32), sds((B, N, C), f32)],
        scratch_shapes=[pltpu.VMEM((N, N), f32), pltpu.VMEM((N, N), f32),
                        pltpu.VMEM((N, N), f32), pltpu.VMEM((N, N), f32),
                        pltpu.VMEM((N, 256), f32), pltpu.VMEM((N, 256), jnp.int32)],
        interpret=_INTERPRET,
    )(*ins)

    embeddings = jnp.transpose(
        jnp.concatenate([out1, out2, out3, out4, out5], axis=2), (0, 2, 1))
    out5_t = jnp.transpose(out5, (0, 2, 1))
    lf2_f = lf2.reshape(B * N, C)
    probs_f = probs.reshape(B * N, C)
    return (embeddings, out5_t, lf2_f, probs_f, probs_f)


# speedup vs baseline: 3.3448x; 3.3448x over previous
import jax
import jax.numpy as jnp
from jax.experimental import pallas as pl
from jax.experimental.pallas import tpu as pltpu

B, N, K, C = 4, 1024, 20, 50
BN_EPS = 1e-5


def _mmT(a, w):
    return jax.lax.dot_general(
        a.astype(jnp.bfloat16), w.astype(jnp.bfloat16),
        (((1,), (1,)), ((), ())), preferred_element_type=jnp.float32)


def _mm(a, b):
    return jax.lax.dot_general(
        a.astype(jnp.bfloat16), b.astype(jnp.bfloat16),
        (((1,), (0,)), ((), ())), preferred_element_type=jnp.float32)


def _bn_s(g):
    return g / jnp.sqrt(jnp.asarray(1.0 + BN_EPS, jnp.float32))


def _cbr(h, W, cb, g, bb, relu=True):
    y = (_mmT(h, W) + cb) * _bn_s(g) + bb
    return jnp.maximum(y, 0.0) if relu else y


def _argmax_lanes(v):
    n, c = v.shape
    m = jnp.max(v, axis=1, keepdims=True)
    iota = jax.lax.broadcasted_iota(jnp.int32, (n, c), 1)
    cand = jnp.where(v == m, iota, c)
    return jnp.min(cand, axis=1, keepdims=True)


def _tnet_conv(h, ref, pref):
    a = _cbr(h, ref[pref + 'c1W'][...], ref[pref + 'c1b'][...],
             ref[pref + 'bn1g'][...], ref[pref + 'bn1b'][...])
    a = _cbr(a, ref[pref + 'c2W'][...], ref[pref + 'c2b'][...],
             ref[pref + 'bn2g'][...], ref[pref + 'bn2b'][...])
    a = _cbr(a, ref[pref + 'c3W'][...], ref[pref + 'c3b'][...],
             ref[pref + 'bn3g'][...], ref[pref + 'bn3b'][...])
    return jnp.max(a, axis=0)


def _p1_kernel(refs_by_name, x_ref, tmax_ref):
    h = x_ref[0]
    tmax_ref[0, 0, :] = _tnet_conv(h, refs_by_name, 't3_')


def _fc_head(t, ref, pref, eye_flat):
    h = jnp.maximum((_mmT(t, ref[pref + 'f1W'][...]) + ref[pref + 'f1b'][...])
                    * _bn_s(ref[pref + 'fbn1g'][...]) + ref[pref + 'fbn1b'][...], 0.0)
    h = jnp.maximum((_mmT(h, ref[pref + 'f2W'][...]) + ref[pref + 'f2b'][...])
                    * _bn_s(ref[pref + 'fbn2g'][...]) + ref[pref + 'fbn2b'][...], 0.0)
    h = _mmT(h, ref[pref + 'f3W'][...]) + ref[pref + 'f3b'][...]
    return h + eye_flat


def _p2_kernel(refs_by_name, tmax_ref, out_ref):
    iota = jax.lax.broadcasted_iota(jnp.int32, (B, 9), 1)
    eye_flat = jnp.where((iota == 0) | (iota == 4) | (iota == 8), 1.0, 0.0)
    out_ref[...] = _fc_head(tmax_ref[:, 0, :], refs_by_name, 't3_', eye_flat)


def _p4_kernel(refs_by_name, tmax_ref, out_ref):
    k2 = out_ref.shape[1]
    iota = jax.lax.broadcasted_iota(jnp.int32, (B, k2), 1)
    eye_flat = jnp.where((iota >> 7) == (iota & 127), 1.0, 0.0)
    out_ref[...] = _fc_head(tmax_ref[:, 0, :], refs_by_name, 'tk_', eye_flat)


def _p3_kernel(refs_by_name, x_ref, tr_ref, out1_ref, out2_ref, out3_ref, tmax2_ref):
    h = _mm(x_ref[0], tr_ref[0])
    o1 = _cbr(h, refs_by_name['c1W'][...], refs_by_name['c1b'][...],
              refs_by_name['c1g'][...], refs_by_name['c1bb'][...])
    o2 = _cbr(o1, refs_by_name['c2W'][...], refs_by_name['c2b'][...],
              refs_by_name['c2g'][...], refs_by_name['c2bb'][...])
    o3 = _cbr(o2, refs_by_name['c3W'][...], refs_by_name['c3b'][...],
              refs_by_name['c3g'][...], refs_by_name['c3bb'][...])
    out1_ref[0] = o1
    out2_ref[0] = o2
    out3_ref[0] = o3
    tmax2_ref[0, 0, :] = _tnet_conv(o3, refs_by_name, 'tk_')


def _p5_kernel(refs_by_name, out3_ref, trkd_ref, out4_ref, out5_ref, lf_ref):
    h = _mm(out3_ref[0], trkd_ref[0])
    o4 = _cbr(h, refs_by_name['c4W'][...], refs_by_name['c4b'][...],
              refs_by_name['c4g'][...], refs_by_name['c4bb'][...])
    o5 = _cbr(o4, refs_by_name['c5W'][...], refs_by_name['c5b'][...],
              refs_by_name['c5g'][...], refs_by_name['c5bb'][...], relu=False)
    hc = _cbr(o5, refs_by_name['k1W'][...], refs_by_name['k1b'][...],
              refs_by_name['k1g'][...], refs_by_name['k1bb'][...])
    hc = _cbr(hc, refs_by_name['k2W'][...], refs_by_name['k2b'][...],
              refs_by_name['k2g'][...], refs_by_name['k2bb'][...])
    lf = _mmT(hc, refs_by_name['k3W'][...]) + refs_by_name['k3b'][...]
    out4_ref[0] = o4
    out5_ref[0] = o5
    lf_ref[0] = lf


def _knn_accum(f, dd_ref, mask_ref):
    sq = jnp.sum(f * f, axis=1, keepdims=True)
    ff = _mmT(f, f)
    d = sq + jnp.transpose(sq) - 2.0 * ff
    iota_r = jax.lax.broadcasted_iota(jnp.int32, (N, N), 0)
    iota_c = jax.lax.broadcasted_iota(jnp.int32, (N, N), 1)
    dd_ref[...] = jnp.where(iota_r == iota_c, jnp.inf, d)

    def body(_, carry):
        dd = dd_ref[...]
        m = jnp.min(dd, axis=1, keepdims=True)
        cand = jnp.where(dd == m, iota_c, N)
        j = jnp.min(cand, axis=1, keepdims=True)
        dd_ref[...] = jnp.where(iota_c == j, jnp.inf, dd)
        return carry

    jax.lax.fori_loop(0, K, body, 0)
    sel_mask = jnp.logical_and(jnp.isinf(dd_ref[...]), iota_r != iota_c)
    mask_ref[...] = jnp.where(sel_mask, 1.0, mask_ref[...])


def _p6_kernel(x_ref, out5_ref, lf_ref, nrm_ref, gcnW_ref, gcnb_ref,
               lf2_ref, probs_ref, dd_ref, mask_ref, w_ref):
    mask_ref[...] = jnp.zeros((N, N), jnp.float32)
    _knn_accum(x_ref[0], dd_ref, mask_ref)
    _knn_accum(out5_ref[0], dd_ref, mask_ref)
    mq = mask_ref[...]

    lf = lf_ref[0]
    label = _argmax_lanes(lf)
    iota_cls = jax.lax.broadcasted_iota(jnp.int32, (N, C), 1)
    oh = jnp.where(iota_cls == label, 1.0, 0.0)
    votes = _mm(mq, oh)
    freq = _argmax_lanes(votes)

    vm = jnp.logical_and(mq > 0.0, jnp.transpose(label) == freq)

    nf = nrm_ref[0]
    n0, n1, n2 = nf[:, 0:1], nf[:, 1:2], nf[:, 2:3]
    num = (n0 * jnp.transpose(n0) + n1 * jnp.transpose(n1)
           + n2 * jnp.transpose(n2))
    nrm = jnp.maximum(jnp.sqrt(n0 * n0 + n1 * n1 + n2 * n2), 1e-8)
    ew = jnp.abs(num / (nrm * jnp.transpose(nrm)))

    iota_r = jax.lax.broadcasted_iota(jnp.int32, (N, N), 0)
    iota_c = jax.lax.broadcasted_iota(jnp.int32, (N, N), 1)
    wd = jnp.where(vm, ew, 0.0) + jnp.where(iota_r == iota_c, 1.0, 0.0)
    w_ref[...] = wd
    deg = jnp.sum(wd, axis=1, keepdims=True)
    dinv = jnp.where(deg > 0.0, 1.0 / jnp.sqrt(deg), 0.0)
    wn = (w_ref[...] * jnp.transpose(dinv)) * dinv

    xw = _mmT(lf, gcnW_ref[...])
    out = jax.lax.dot_general(wn, xw, (((1,), (0,)), ((), ())),
                              preferred_element_type=jnp.float32,
                              precision=jax.lax.Precision.HIGHEST)
    lf2 = out + gcnb_ref[...]
    lf2_ref[0] = lf2
    fin = _argmax_lanes(lf2)
    probs_ref[0] = jnp.where(iota_cls == fin, 1.0, 0.0)


_P1_NAMES = tuple('t3_' + s for s in (
    'c1W', 'c1b', 'bn1g', 'bn1b', 'c2W', 'c2b', 'bn2g', 'bn2b',
    'c3W', 'c3b', 'bn3g', 'bn3b'))
_P2_NAMES = tuple('t3_' + s for s in (
    'f1W', 'f1b', 'fbn1g', 'fbn1b', 'f2W', 'f2b', 'fbn2g', 'fbn2b', 'f3W', 'f3b'))
_P3_NAMES = ('c1W', 'c1b', 'c1g', 'c1bb', 'c2W', 'c2b', 'c2g', 'c2bb',
             'c3W', 'c3b', 'c3g', 'c3bb') + tuple('tk_' + s for s in (
    'c1W', 'c1b', 'bn1g', 'bn1b', 'c2W', 'c2b', 'bn2g', 'bn2b',
    'c3W', 'c3b', 'bn3g', 'bn3b'))
_P4_NAMES = tuple('tk_' + s for s in (
    'f1W', 'f1b', 'fbn1g', 'fbn1b', 'f2W', 'f2b', 'fbn2g', 'fbn2b', 'f3W', 'f3b'))
_P5_NAMES = ('c4W', 'c4b', 'c4g', 'c4bb', 'c5W', 'c5b', 'c5g', 'c5bb',
             'k1W', 'k1b', 'k1g', 'k1bb', 'k2W', 'k2b', 'k2g', 'k2bb',
             'k3W', 'k3b')


def _prep(p, names):
    out = {}
    for nm in names:
        v = p[nm]
        out[nm] = v.reshape(1, -1) if v.ndim == 1 else v
    return out


def _named_kernel(body, names):
    def k(*refs):
        by_name = dict(zip(names, refs[:len(names)]))
        body(by_name, *refs[len(names):])
    return k


def _full_spec(v):
    nd = v.ndim
    return pl.BlockSpec(v.shape, lambda *_: (0,) * nd)


def _batch_spec(shape):
    return pl.BlockSpec((1,) + shape[1:], lambda b: (b,) + (0,) * (len(shape) - 1))


def _call_grid(body, names, wdict, ins, outs):
    wvals = [wdict[nm] for nm in names]
    return pl.pallas_call(
        _named_kernel(body, names),
        grid=(B,),
        in_specs=[_full_spec(v) for v in wvals] + [_batch_spec(v.shape) for v in ins],
        out_specs=[_batch_spec(s.shape) for s in outs],
        out_shape=outs,

    )(*wvals, *ins)


def _call_flat(body, names, wdict, ins, outs):
    wvals = [wdict[nm] for nm in names]
    return pl.pallas_call(
        _named_kernel(body, names),
        in_specs=[_full_spec(v) for v in wvals] + [_full_spec(v) for v in ins],
        out_specs=[_full_spec(s) for s in outs],
        out_shape=outs,

    )(*wvals, *ins)


def kernel(x, normals, params):
    p = params
    f32 = jnp.float32
    sds = jax.ShapeDtypeStruct

    (tmax,) = _call_grid(_p1_kernel, _P1_NAMES, _prep(p, _P1_NAMES), [x],
                         [sds((B, 1, 1024), f32)])
    (tr9,) = _call_flat(_p2_kernel, _P2_NAMES, _prep(p, _P2_NAMES), [tmax],
                        [sds((B, 9), f32)])
    trans3d = tr9.reshape(B, 3, 3)
    out1, out2, out3, tmax2 = _call_grid(
        _p3_kernel, _P3_NAMES, _prep(p, _P3_NAMES), [x, trans3d],
        [sds((B, N, 64), f32), sds((B, N, 128), f32), sds((B, N, 128), f32),
         sds((B, 1, 1024), f32)])
    (trkd,) = _call_flat(_p4_kernel, _P4_NAMES, _prep(p, _P4_NAMES), [tmax2],
                         [sds((B, 16384), f32)])
    transkd = trkd.reshape(B, 128, 128)
    out4, out5, lf = _call_grid(
        _p5_kernel, _P5_NAMES, _prep(p, _P5_NAMES), [out3, transkd],
        [sds((B, N, 256), f32), sds((B, N, 512), f32), sds((B, N, C), f32)])
    gcnW = p['gcnW']
    gcnb = p['gcnb'].reshape(1, C)
    ins = [x, out5, lf, normals, gcnW, gcnb]
    lf2, probs = pl.pallas_call(
        _p6_kernel,
        grid=(B,),
        in_specs=[_batch_spec(x.shape), _batch_spec(out5.shape),
                  _batch_spec(lf.shape), _batch_spec(normals.shape),
                  _full_spec(gcnW), _full_spec(gcnb)],
        out_specs=[_batch_spec((B, N, C)), _batch_spec((B, N, C))],
        out_shape=[sds((B, N, C), f32), sds((B, N, C), f32)],
        scratch_shapes=[pltpu.VMEM((N, N), f32), pltpu.VMEM((N, N), f32),
                        pltpu.VMEM((N, N), f32)],

    )(*ins)

    embeddings = jnp.transpose(
        jnp.concatenate([out1, out2, out3, out4, out5], axis=2), (0, 2, 1))
    out5_t = jnp.transpose(out5, (0, 2, 1))
    lf2_f = lf2.reshape(B * N, C)
    probs_f = probs.reshape(B * N, C)
    return (embeddings, out5_t, lf2_f, probs_f, probs_f)
